# baseline (speedup 1.0000x reference)
import functools

import jax
import jax.numpy as jnp
from jax import lax
from jax.experimental import pallas as pl
from jax.experimental.pallas import tpu as pltpu
from jax.experimental.pallas import tpu_sc as plsc

NC = 2
NS = 16
NW = NC * NS
A = 10000
A_PAD = 10240
D = 128
PAIRS = 320000
C = 80
N_LOADS = PAIRS // (NW * C)
PW = PAIRS // NW
ROWS_PER_TILE = A_PAD // NS
NBUF = 4

_mesh = plsc.VectorSubcoreMesh(
    core_axis_name="c", subcore_axis_name="s", num_cores=NC, num_subcores=NS
)


@functools.partial(
    pl.kernel,
    out_type=jax.ShapeDtypeStruct((NC, A_PAD, D), jnp.float32),
    mesh=_mesh,
    scratch_types=[
        pltpu.VMEM((C,), jnp.int32),
        pltpu.VMEM((C,), jnp.int32),
        pltpu.VMEM((C,), jnp.int32),
        pltpu.VMEM((C,), jnp.int32),
        pltpu.VMEM((NBUF, C, D), jnp.float32),
        pltpu.VMEM_SHARED((A_PAD, D), jnp.float32),
        pltpu.SemaphoreType.DMA((NBUF,)),
        pltpu.SemaphoreType.DMA((NBUF,)),
        pltpu.SemaphoreType.DMA((NBUF,)),
        pltpu.SemaphoreType.DMA,
    ],
)
def _scatter_partials(idx_hbm, inter_hbm, out_hbm,
                      idx0, idx1, idx2, idx3, rows_v, acc_sh,
                      lsem, isem, ssem, zsem):
    idx_slots = (idx0, idx1, idx2, idx3)
    c = lax.axis_index("c")
    s = lax.axis_index("s")
    w = s * NC + c

    def load_desc(i, b, q=0):
        return pltpu.make_async_copy(
            inter_hbm.at[pl.ds(w * PW + i * C, C)], rows_v.at[b], lsem.at[b])

    def idx_desc(i, b, q):
        return pltpu.make_async_copy(
            idx_hbm.at[pl.ds(w * PW + i * C, C)], idx_slots[q], isem.at[b])

    def scat_desc(i, b, q=0):
        return pltpu.make_async_copy(
            rows_v.at[b], acc_sh.at[idx_slots[q]], ssem.at[b])

    def start_loads(i, b):
        for q in range(NBUF):
            @pl.when(b == q)
            def _(q=q):
                idx_desc(i, b, q).start()
                load_desc(i, b, q).start()

    start_loads(0, 0)
    start_loads(1, 1)

    zval = jnp.zeros((16,), jnp.float32)

    def zrow(r, carry):
        for g in range(D // 16):
            rows_v[NBUF - 1, r, pl.ds(g * 16, 16)] = zval
        return carry

    lax.fori_loop(0, C, zrow, 0)
    n_zcopies = ROWS_PER_TILE // C
    for t in range(n_zcopies):
        pltpu.async_copy(
            rows_v.at[NBUF - 1],
            acc_sh.at[pl.ds(s * ROWS_PER_TILE + t * C, C)], zsem)
    for t in range(n_zcopies):
        pltpu.make_async_copy(
            rows_v.at[NBUF - 1],
            acc_sh.at[pl.ds(s * ROWS_PER_TILE, C)], zsem).wait()
    plsc.subcore_barrier()

    def body(i, carry):
        b = lax.rem(i, NBUF)
        idx_desc(i, b, 0).wait()
        load_desc(i, b).wait()
        for q in range(NBUF):
            @pl.when(b == q)
            def _(q=q):
                pltpu.async_copy(rows_v.at[q], acc_sh.at[idx_slots[q]],
                                 ssem.at[q], add=True)
        @pl.when(i + 2 < N_LOADS)
        def _():
            nb = lax.rem(i + 2, NBUF)
            @pl.when(i >= 2)
            def _():
                scat_desc(i - 2, nb).wait()
            start_loads(i + 2, nb)
        return carry

    lax.fori_loop(0, N_LOADS, body, 0)

    for t in (4, 3, 2, 1):
        scat_desc(N_LOADS - t, lax.rem(N_LOADS - t, NBUF)).wait()

    plsc.subcore_barrier()
    stripe = pl.ds(s * ROWS_PER_TILE, ROWS_PER_TILE)
    pltpu.sync_copy(acc_sh.at[stripe], out_hbm.at[c, stripe])


def _merge_body(p_ref, o_ref):
    o_ref[...] = p_ref[0] + p_ref[1]


_MERGE_ROWS = 2000


def _merge(partials):
    return pl.pallas_call(
        _merge_body,
        grid=(A // _MERGE_ROWS,),
        in_specs=[pl.BlockSpec((NC, _MERGE_ROWS, D), lambda i: (0, i, 0))],
        out_specs=pl.BlockSpec((_MERGE_ROWS, D), lambda i: (i, 0)),
        out_shape=jax.ShapeDtypeStruct((A, D), jnp.float32),
    )(partials)


def kernel(ind_2, prop, inter):
    idx = ind_2[:, 0].astype(jnp.int32)
    partials = _scatter_partials(idx, inter)
    return _merge(partials)
